# baseline (speedup 1.0000x reference)
import functools

import jax
import jax.numpy as jnp
from jax.experimental import pallas as pl
from jax.experimental.pallas import tpu as pltpu
from jax.experimental.pallas import tpu_sc as plsc

H = 128
W = 128
ALPHA_T = 1.0 / 255.0
ROWS = 8
UNROLL = 16
SUPER = 16
T_EPS = 1e-4
HALF = (ROWS - 1) * 0.5
LANES = 16


def _make_compact(g, half_n, cap):
    n_iter = half_n // LANES
    mesh = plsc.VectorSubcoreMesh(core_axis_name="c", subcore_axis_name="s")

    @functools.partial(
        pl.kernel, mesh=mesh,
        compiler_params=pltpu.CompilerParams(needs_layout_passes=False),
        out_type=[
            jax.ShapeDtypeStruct((32, cap), jnp.int32),
            jax.ShapeDtypeStruct((32, LANES), jnp.int32),
        ],
        scratch_types=[
            pltpu.VMEM((half_n,), jnp.int32),
            pltpu.VMEM((g + 8,), jnp.float32),
            pltpu.VMEM((g + 8,), jnp.float32),
            pltpu.VMEM((cap,), jnp.int32),
            pltpu.VMEM((LANES,), jnp.int32),
        ],
    )
    def compact(order_hbm, my_hbm, ye_hbm, lists_hbm, counts_hbm,
                ord_v, my_v, ye_v, list_v, cnt_v):
        band = jax.lax.axis_index("s")
        half = jax.lax.axis_index("c")
        wid = band * 2 + half
        pltpu.sync_copy(order_hbm.at[pl.ds(half * half_n, half_n)], ord_v)
        pltpu.sync_copy(my_hbm, my_v)
        pltpu.sync_copy(ye_hbm, ye_v)

        yc = jnp.zeros((LANES,), jnp.float32) + (
            band.astype(jnp.float32) * ROWS + 0.5 + HALF)

        def step(i, cnt):
            ordv = ord_v[pl.ds(i * LANES, LANES)]
            myv = plsc.load_gather(my_v, [ordv])
            yev = plsc.load_gather(ye_v, [ordv])
            hit = jnp.abs(myv - yc) <= yev + HALF
            incl = jnp.cumsum(hit.astype(jnp.int32))
            pos = cnt + incl - 1
            plsc.store_scatter(list_v, [pos], ordv, mask=hit)
            return cnt + plsc.all_reduce_population_count(hit)

        cnt = jax.lax.fori_loop(0, n_iter, step,
                                jnp.zeros((LANES,), jnp.int32))

        pos = cnt + jax.lax.iota(jnp.int32, LANES)
        plsc.store_scatter(list_v, [pos],
                           jnp.zeros((LANES,), jnp.int32) + g,
                           mask=pos < cap)
        cnt_v[...] = cnt
        pltpu.sync_copy(list_v, lists_hbm.at[wid])
        pltpu.sync_copy(cnt_v, counts_hbm.at[wid])

    return compact


def _raster_body(lists_ref, counts_ref, mx_ref, my_ref, na_ref, nb_ref,
                 nc_ref, opa_ref, cr_ref, cg_ref, cbl_ref,
                 outr_ref, outg_ref, outb_ref, t_ref):
    band = pl.program_id(0)
    y0 = band.astype(jnp.float32) * ROWS + 0.5
    ys = jax.lax.broadcasted_iota(jnp.int32, (ROWS, W), 0).astype(jnp.float32) + y0
    xs = jax.lax.broadcasted_iota(jnp.int32, (ROWS, W), 1).astype(jnp.float32) + 0.5

    t_ref[...] = jnp.ones((ROWS, W), jnp.float32)
    zero = jnp.zeros((ROWS, W), jnp.float32)
    outr_ref[...] = zero
    outg_ref[...] = zero
    outb_ref[...] = zero

    for h in range(2):
        cnt = counts_ref[band, h, 0]
        nblk = (cnt + (UNROLL - 1)) // UNROLL

        def one_block(bi, _):
            base = bi * UNROLL
            idx = [lists_ref[0, h, base + k] for k in range(UNROLL)]
            trans = t_ref[...]
            accr = outr_ref[...]
            accg = outg_ref[...]
            accb = outb_ref[...]
            for s in range(UNROLL // 8):
                ids = idx[s * 8:(s + 1) * 8]
                al = []
                for i in ids:
                    dx = xs - mx_ref[i]
                    dy = ys - my_ref[i]
                    q = (na_ref[i] * dx * dx
                         + nb_ref[i] * dx * dy
                         + nc_ref[i] * dy * dy)
                    a_k = opa_ref[i] * jnp.exp(q)
                    a_k = jnp.minimum(a_k, 0.999)
                    al.append(jnp.where(a_k >= ALPHA_T, a_k, 0.0))
                om = [1.0 - a_k for a_k in al]
                om01 = om[0] * om[1]
                om23 = om[2] * om[3]
                om45 = om[4] * om[5]
                om67 = om[6] * om[7]
                om03 = om01 * om23
                om47 = om45 * om67
                p = [None, om[0], om01, om01 * om[2], om03,
                     om03 * om[4], om03 * om45, (om03 * om45) * om[6]]
                wg = [al[0]] + [al[k] * p[k] for k in range(1, 8)]
                for acc_i, cref in ((0, cr_ref), (1, cg_ref), (2, cbl_ref)):
                    s_c = (((wg[0] * cref[ids[0]] + wg[1] * cref[ids[1]])
                            + (wg[2] * cref[ids[2]] + wg[3] * cref[ids[3]]))
                           + ((wg[4] * cref[ids[4]] + wg[5] * cref[ids[5]])
                              + (wg[6] * cref[ids[6]] + wg[7] * cref[ids[7]])))
                    if acc_i == 0:
                        accr = accr + trans * s_c
                    elif acc_i == 1:
                        accg = accg + trans * s_c
                    else:
                        accb = accb + trans * s_c
                trans = trans * (om03 * om47)
            outr_ref[...] = accr
            outg_ref[...] = accg
            outb_ref[...] = accb
            t_ref[...] = trans
            return 0

        def sb_cond(carry):
            b0, live = carry
            return jnp.logical_and(live, b0 < nblk)

        def sb_body(carry):
            b0, _ = carry
            hi = jnp.minimum(b0 + SUPER, nblk)
            jax.lax.fori_loop(b0, hi, one_block, 0)
            live = jnp.max(t_ref[...]) >= T_EPS
            return hi, live

        jax.lax.while_loop(sb_cond, sb_body,
                           (jnp.int32(0), jnp.bool_(True)))


def _rasterize(lists3, counts3, mx, my, na, nb, nc, opa, cr, cg, cbl,
               interpret=False):
    cap = lists3.shape[2]
    smem = pl.BlockSpec(memory_space=pltpu.SMEM)
    outs = jax.ShapeDtypeStruct((H, W), jnp.float32)
    return pl.pallas_call(
        _raster_body,
        grid=(H // ROWS,),
        in_specs=[pl.BlockSpec((1, 2, cap), lambda i: (i, 0, 0),
                               memory_space=pltpu.SMEM)] + [smem] * 10,
        out_specs=[pl.BlockSpec((ROWS, W), lambda i: (i, 0))] * 3,
        out_shape=[outs, outs, outs],
        scratch_shapes=[pltpu.VMEM((ROWS, W), jnp.float32)],
        interpret=interpret,
    )(lists3, counts3, mx, my, na, nb, nc, opa, cr, cg, cbl)


def kernel(means2d, conics, colors, opacities, depths):
    b = means2d.shape[0]
    g = means2d.shape[1]
    order = jnp.argsort(depths, axis=1).astype(jnp.int32)

    a = conics[..., 0]
    bb = conics[..., 1]
    c = conics[..., 2]
    det = a * c - bb * bb
    qm = 2.0 * jnp.log(255.0 * opacities)
    ry = jnp.sqrt(jnp.maximum(qm, 0.0) * a / det)
    yext = jnp.where(opacities >= ALPHA_T, ry, -1e9).astype(jnp.float32)

    half_n = -(-g // (2 * LANES)) * LANES
    cap = half_n + LANES
    compact = _make_compact(g, half_n, cap)

    def padded(x, value, n):
        return jnp.concatenate([x, jnp.full((n - x.shape[0],), value, x.dtype)])

    imgs = []
    for i in range(b):
        order_i = padded(order[i], g, 2 * half_n)
        my_i = padded(means2d[i, :, 1], 0.0, g + 8)
        ye_i = padded(yext[i], -1e9, g + 8)
        lists, counts = compact(order_i, my_i, ye_i)
        r, gg, bl = _rasterize(
            lists.reshape(H // ROWS, 2, cap),
            counts.reshape(H // ROWS, 2, LANES),
            padded(means2d[i, :, 0], 0.0, g + 8),
            my_i,
            padded(-0.5 * conics[i, :, 0], 0.0, g + 8),
            padded(-conics[i, :, 1], 0.0, g + 8),
            padded(-0.5 * conics[i, :, 2], 0.0, g + 8),
            padded(opacities[i], 0.0, g + 8),
            padded(colors[i, :, 0], 0.0, g + 8),
            padded(colors[i, :, 1], 0.0, g + 8),
            padded(colors[i, :, 2], 0.0, g + 8),
        )
        imgs.append(jnp.stack([r, gg, bl], axis=-1))
    return jnp.stack(imgs, axis=0)


# speedup vs baseline: 1.0640x; 1.0640x over previous
import functools

import jax
import jax.numpy as jnp
from jax.experimental import pallas as pl
from jax.experimental.pallas import tpu as pltpu
from jax.experimental.pallas import tpu_sc as plsc

H = 128
W = 128
ALPHA_T = 1.0 / 255.0
ROWS = 8
UNROLL = 16
SUPER = 16
T_EPS = 1e-4
HALF = (ROWS - 1) * 0.5
LANES = 16


def _make_compact(g, half_n, cap):
    n_iter = half_n // LANES
    mesh = plsc.VectorSubcoreMesh(core_axis_name="c", subcore_axis_name="s")

    @functools.partial(
        pl.kernel, mesh=mesh,
        compiler_params=pltpu.CompilerParams(needs_layout_passes=False),
        out_type=[
            jax.ShapeDtypeStruct((32, cap), jnp.int32),
            jax.ShapeDtypeStruct((32, LANES), jnp.int32),
        ],
        scratch_types=[
            pltpu.VMEM((half_n,), jnp.int32),
            pltpu.VMEM((g + 8,), jnp.float32),
            pltpu.VMEM((g + 8,), jnp.float32),
            pltpu.VMEM((cap,), jnp.int32),
            pltpu.VMEM((LANES,), jnp.int32),
        ],
    )
    def compact(order_hbm, my_hbm, ye_hbm, lists_hbm, counts_hbm,
                ord_v, my_v, ye_v, list_v, cnt_v):
        band = jax.lax.axis_index("s")
        half = jax.lax.axis_index("c")
        wid = band * 2 + half
        pltpu.sync_copy(order_hbm.at[pl.ds(half * half_n, half_n)], ord_v)
        pltpu.sync_copy(my_hbm, my_v)
        pltpu.sync_copy(ye_hbm, ye_v)

        yc = jnp.zeros((LANES,), jnp.float32) + (
            band.astype(jnp.float32) * ROWS + 0.5 + HALF)

        def step(i, cnt):
            ordv = ord_v[pl.ds(i * LANES, LANES)]
            myv = plsc.load_gather(my_v, [ordv])
            yev = plsc.load_gather(ye_v, [ordv])
            hit = jnp.abs(myv - yc) <= yev + HALF
            incl = jnp.cumsum(hit.astype(jnp.int32))
            pos = cnt + incl - 1
            plsc.store_scatter(list_v, [pos], ordv, mask=hit)
            return cnt + plsc.all_reduce_population_count(hit)

        cnt = jax.lax.fori_loop(0, n_iter, step,
                                jnp.zeros((LANES,), jnp.int32))

        pos = cnt + jax.lax.iota(jnp.int32, LANES)
        plsc.store_scatter(list_v, [pos],
                           jnp.zeros((LANES,), jnp.int32) + g,
                           mask=pos < cap)
        cnt_v[...] = cnt
        pltpu.sync_copy(list_v, lists_hbm.at[wid])
        pltpu.sync_copy(cnt_v, counts_hbm.at[wid])

    return compact


def _raster_body(lists_ref, counts_ref, mx_ref, my_ref, na_ref, nb_ref,
                 nc_ref, opa_ref, cr_ref, cg_ref, cbl_ref,
                 outr_ref, outg_ref, outb_ref, t_ref):
    band = pl.program_id(0)
    y0 = band.astype(jnp.float32) * ROWS + 0.5
    ys = jax.lax.broadcasted_iota(jnp.int32, (ROWS, W), 0).astype(jnp.float32) + y0
    xs = jax.lax.broadcasted_iota(jnp.int32, (ROWS, W), 1).astype(jnp.float32) + 0.5

    t_ref[...] = jnp.ones((ROWS, W), jnp.float32)
    zero = jnp.zeros((ROWS, W), jnp.float32)
    outr_ref[...] = zero
    outg_ref[...] = zero
    outb_ref[...] = zero

    for h in range(2):
        cnt = counts_ref[band, h, 0]
        nblk = (cnt + (UNROLL - 1)) // UNROLL

        def one_block(bi, _):
            base = bi * UNROLL
            idx = [lists_ref[0, h, base + k] for k in range(UNROLL)]
            trans = t_ref[...]
            accr = outr_ref[...]
            accg = outg_ref[...]
            accb = outb_ref[...]
            for k in range(UNROLL):
                i = idx[k]
                dx = xs - mx_ref[i]
                dy = ys - my_ref[i]
                q = (na_ref[i] * dx * dx
                     + nb_ref[i] * dx * dy
                     + nc_ref[i] * dy * dy)
                alpha = opa_ref[i] * jnp.exp(q)
                alpha = jnp.minimum(alpha, 0.999)
                alpha = jnp.where(alpha >= ALPHA_T, alpha, 0.0)
                wgt = alpha * trans
                accr = accr + wgt * cr_ref[i]
                accg = accg + wgt * cg_ref[i]
                accb = accb + wgt * cbl_ref[i]
                trans = trans * (1.0 - alpha)
            outr_ref[...] = accr
            outg_ref[...] = accg
            outb_ref[...] = accb
            t_ref[...] = trans
            return 0

        jax.lax.fori_loop(jnp.int32(0), nblk, one_block, 0)


def _rasterize(lists3, counts3, mx, my, na, nb, nc, opa, cr, cg, cbl,
               interpret=False):
    cap = lists3.shape[2]
    smem = pl.BlockSpec(memory_space=pltpu.SMEM)
    outs = jax.ShapeDtypeStruct((H, W), jnp.float32)
    return pl.pallas_call(
        _raster_body,
        grid=(H // ROWS,),
        in_specs=[pl.BlockSpec((1, 2, cap), lambda i: (i, 0, 0),
                               memory_space=pltpu.SMEM)] + [smem] * 10,
        out_specs=[pl.BlockSpec((ROWS, W), lambda i: (i, 0))] * 3,
        out_shape=[outs, outs, outs],
        scratch_shapes=[pltpu.VMEM((ROWS, W), jnp.float32)],
        interpret=interpret,
    )(lists3, counts3, mx, my, na, nb, nc, opa, cr, cg, cbl)


def kernel(means2d, conics, colors, opacities, depths):
    b = means2d.shape[0]
    g = means2d.shape[1]
    order = jnp.argsort(depths, axis=1).astype(jnp.int32)

    a = conics[..., 0]
    bb = conics[..., 1]
    c = conics[..., 2]
    det = a * c - bb * bb
    qm = 2.0 * jnp.log(255.0 * opacities)
    ry = jnp.sqrt(jnp.maximum(qm, 0.0) * a / det)
    yext = jnp.where(opacities >= ALPHA_T, ry, -1e9).astype(jnp.float32)

    half_n = -(-g // (2 * LANES)) * LANES
    cap = half_n + LANES
    compact = _make_compact(g, half_n, cap)

    def padded(x, value, n):
        return jnp.concatenate([x, jnp.full((n - x.shape[0],), value, x.dtype)])

    imgs = []
    for i in range(b):
        order_i = padded(order[i], g, 2 * half_n)
        my_i = padded(means2d[i, :, 1], 0.0, g + 8)
        ye_i = padded(yext[i], -1e9, g + 8)
        lists, counts = compact(order_i, my_i, ye_i)
        r, gg, bl = _rasterize(
            lists.reshape(H // ROWS, 2, cap),
            counts.reshape(H // ROWS, 2, LANES),
            padded(means2d[i, :, 0], 0.0, g + 8),
            my_i,
            padded(-0.5 * conics[i, :, 0], 0.0, g + 8),
            padded(-conics[i, :, 1], 0.0, g + 8),
            padded(-0.5 * conics[i, :, 2], 0.0, g + 8),
            padded(opacities[i], 0.0, g + 8),
            padded(colors[i, :, 0], 0.0, g + 8),
            padded(colors[i, :, 1], 0.0, g + 8),
            padded(colors[i, :, 2], 0.0, g + 8),
        )
        imgs.append(jnp.stack([r, gg, bl], axis=-1))
    return jnp.stack(imgs, axis=0)
